# baseline (speedup 1.0000x reference)
import functools

import jax
import jax.numpy as jnp
from jax import lax
from jax.experimental import pallas as pl
from jax.experimental.pallas import tpu as pltpu
from jax.experimental.pallas import tpu_sc as plsc

N = 2048
NC = 1
NS = 16
NW = NC * NS
L = 16
CHUNK = N // NW
KVECS = N // L
IB = 8
LN2 = 0.6931471805599453


def _sum_all(x, iota):
    for sh in (8, 4, 2, 1):
        x = x + jnp.take(x, iota ^ sh)
    return x


def _max_all(x, iota):
    for sh in (8, 4, 2, 1):
        x = jnp.maximum(x, jnp.take(x, iota ^ sh))
    return x


def _vlog(x):
    bits = plsc.bitcast(x, jnp.int32)
    ex = (bits >> 23) - 127
    mant = plsc.bitcast((bits & 0x007FFFFF) | 0x3F800000, jnp.float32)
    big = mant > 1.4142135623730951
    mant = jnp.where(big, mant * 0.5, mant)
    ex_f = ex.astype(jnp.float32) + jnp.where(big, 1.0, 0.0)
    z = (mant - 1.0) / (mant + 1.0)
    z2 = z * z
    p = z * (2.0 + z2 * (0.66666667 + z2 * (0.4 + z2 * (0.28571429 + z2 * 0.22222222))))
    return ex_f * LN2 + p


def _body(scores_hbm, labels_hbm, out_hbm, sv, lv, ev, uav, umv, outv):
    wid = lax.axis_index("s") * NC + lax.axis_index("c")
    pltpu.sync_copy(scores_hbm, sv)
    pltpu.sync_copy(labels_hbm, lv)

    iota = lax.iota(jnp.int32, L)

    def max_step(kb, acc):
        return jnp.maximum(acc, sv[pl.ds(kb * L, L)])
    mv = _max_all(lax.fori_loop(0, KVECS, max_step, jnp.full((L,), -3.0e38, jnp.float32),
                                unroll=8), iota)

    def pre_step(kb, _):
        off = kb * L
        ev[pl.ds(off, L)] = jnp.exp(sv[pl.ds(off, L)] - mv)
        uav[pl.ds(off, L)] = plsc.bitcast(lv[pl.ds(off, L)], jnp.int32) * 2
        return 0
    lax.fori_loop(0, KVECS, pre_step, 0, unroll=8)

    base = wid * CHUNK
    tacc = jnp.zeros((L,), jnp.float32)

    for g in range(CHUNK // L):
        gbase = base + g * L
        kbs = gbase >> 4

        def umod_step(kb, _, kbs=kbs):
            off = kb * L
            umv[pl.ds(off, L)] = uav[pl.ds(off, L)] - jnp.where(kb > kbs, 1, 0)
            return 0
        lax.fori_loop(0, KVECS, umod_step, 0, unroll=8)

        l_grp = lv[pl.ds(gbase, L)]
        u_grp = plsc.bitcast(l_grp, jnp.int32) * 2
        ua_str = uav[pl.ds(gbase, L)]
        e_str = ev[pl.ds(gbase, L)]
        kvec_str = iota + gbase
        svec = sv[pl.ds(gbase, L)]

        wvec = jnp.zeros((L,), jnp.float32)
        for ib in range(L // IB):
            u_is = [jnp.take(u_grp, jnp.full((L,), ib * IB + j, jnp.int32))
                    for j in range(IB)]
            i_vals = [gbase + ib * IB + j for j in range(IB)]

            def sweep(kb, accs, u_is=u_is):
                off = kb * L
                um = umv[pl.ds(off, L)]
                ebits = plsc.bitcast(ev[pl.ds(off, L)], jnp.int32)
                return tuple(
                    a + plsc.bitcast(((um - u_is[j]) >> 31) & ebits, jnp.float32)
                    for j, a in enumerate(accs))

            accs = lax.fori_loop(0, KVECS, sweep,
                                 tuple(jnp.zeros((L,), jnp.float32) for _ in range(IB)),
                                 unroll=4)

            for j in range(IB):
                corr = (kvec_str >= i_vals[j]) & (ua_str == u_is[j])
                a = accs[j] + jnp.where(corr, e_str, 0.0)
                wvec = jnp.where(iota == ib * IB + j, _sum_all(a, iota), wvec)

        tacc = tacc + (_vlog(wvec) + mv - svec)

    outv[...] = _sum_all(tacc, iota) * (1.0 / N)
    pltpu.sync_copy(outv, out_hbm.at[wid])


@functools.partial(
    pl.kernel,
    out_type=jax.ShapeDtypeStruct((NW, L), jnp.float32),
    mesh=plsc.VectorSubcoreMesh(
        core_axis_name="c", subcore_axis_name="s", num_cores=NC, num_subcores=NS
    ),
    compiler_params=pltpu.CompilerParams(needs_layout_passes=False),
    scratch_types=[
        pltpu.VMEM((N,), jnp.float32),
        pltpu.VMEM((N,), jnp.float32),
        pltpu.VMEM((N,), jnp.float32),
        pltpu.VMEM((N,), jnp.int32),
        pltpu.VMEM((N,), jnp.int32),
        pltpu.VMEM((L,), jnp.float32),
    ],
)
def _listmle_sc(scores_hbm, labels_hbm, out_hbm, sv, lv, ev, uav, umv, outv):
    _body(scores_hbm, labels_hbm, out_hbm, sv, lv, ev, uav, umv, outv)


def kernel(scores, labels):
    partials = _listmle_sc(scores, labels)
    return jnp.sum(partials[:, 0])


# speedup vs baseline: 1.5869x; 1.5869x over previous
import functools

import jax
import jax.numpy as jnp
from jax import lax
from jax.experimental import pallas as pl
from jax.experimental.pallas import tpu as pltpu
from jax.experimental.pallas import tpu_sc as plsc

N = 2048
NC = 2
NS = 16
NW = NC * NS
L = 16
CHUNK = N // NW
KVECS = N // L
IB = 8
LN2 = 0.6931471805599453


def _sum_all(x, iota):
    for sh in (8, 4, 2, 1):
        x = x + jnp.take(x, iota ^ sh)
    return x


def _max_all(x, iota):
    for sh in (8, 4, 2, 1):
        x = jnp.maximum(x, jnp.take(x, iota ^ sh))
    return x


def _vlog(x):
    bits = plsc.bitcast(x, jnp.int32)
    ex = (bits >> 23) - 127
    mant = plsc.bitcast((bits & 0x007FFFFF) | 0x3F800000, jnp.float32)
    big = mant > 1.4142135623730951
    mant = jnp.where(big, mant * 0.5, mant)
    ex_f = ex.astype(jnp.float32) + jnp.where(big, 1.0, 0.0)
    z = (mant - 1.0) / (mant + 1.0)
    z2 = z * z
    p = z * (2.0 + z2 * (0.66666667 + z2 * (0.4 + z2 * (0.28571429 + z2 * 0.22222222))))
    return ex_f * LN2 + p


def _body(scores_hbm, labels_hbm, out_hbm, sv, lv, ev, uav, umv, outv):
    wid = lax.axis_index("s") * NC + lax.axis_index("c")
    pltpu.sync_copy(scores_hbm, sv)
    pltpu.sync_copy(labels_hbm, lv)

    iota = lax.iota(jnp.int32, L)

    def max_step(kb, acc):
        return jnp.maximum(acc, sv[pl.ds(kb * L, L)])
    mv = _max_all(lax.fori_loop(0, KVECS, max_step, jnp.full((L,), -3.0e38, jnp.float32),
                                unroll=8), iota)

    def pre_step(kb, _):
        off = kb * L
        ev[pl.ds(off, L)] = jnp.exp(sv[pl.ds(off, L)] - mv)
        uav[pl.ds(off, L)] = plsc.bitcast(lv[pl.ds(off, L)], jnp.int32) * 2
        return 0
    lax.fori_loop(0, KVECS, pre_step, 0, unroll=8)

    base = wid * CHUNK
    tacc = jnp.zeros((L,), jnp.float32)

    for g in range(CHUNK // L):
        gbase = base + g * L
        kbs = gbase >> 4

        def umod_step(kb, _, kbs=kbs):
            off = kb * L
            umv[pl.ds(off, L)] = uav[pl.ds(off, L)] - jnp.where(kb > kbs, 1, 0)
            return 0
        lax.fori_loop(0, KVECS, umod_step, 0, unroll=8)

        l_grp = lv[pl.ds(gbase, L)]
        u_grp = plsc.bitcast(l_grp, jnp.int32) * 2
        ua_str = uav[pl.ds(gbase, L)]
        e_str = ev[pl.ds(gbase, L)]
        kvec_str = iota + gbase
        svec = sv[pl.ds(gbase, L)]

        wvec = jnp.zeros((L,), jnp.float32)
        for ib in range(L // IB):
            u_is = [jnp.take(u_grp, jnp.full((L,), ib * IB + j, jnp.int32))
                    for j in range(IB)]
            i_vals = [gbase + ib * IB + j for j in range(IB)]

            def sweep(kb, accs, u_is=u_is):
                off = kb * L
                um = umv[pl.ds(off, L)]
                ebits = plsc.bitcast(ev[pl.ds(off, L)], jnp.int32)
                return tuple(
                    a + plsc.bitcast(((um - u_is[j]) >> 31) & ebits, jnp.float32)
                    for j, a in enumerate(accs))

            accs = lax.fori_loop(0, KVECS, sweep,
                                 tuple(jnp.zeros((L,), jnp.float32) for _ in range(IB)),
                                 unroll=1)

            for j in range(IB):
                corr = (kvec_str >= i_vals[j]) & (ua_str == u_is[j])
                a = accs[j] + jnp.where(corr, e_str, 0.0)
                wvec = jnp.where(iota == ib * IB + j, _sum_all(a, iota), wvec)

        tacc = tacc + (_vlog(wvec) + mv - svec)

    outv[...] = _sum_all(tacc, iota) * (1.0 / N)
    pltpu.sync_copy(outv, out_hbm.at[wid])


@functools.partial(
    pl.kernel,
    out_type=jax.ShapeDtypeStruct((NW, L), jnp.float32),
    mesh=plsc.VectorSubcoreMesh(
        core_axis_name="c", subcore_axis_name="s", num_cores=NC, num_subcores=NS
    ),
    compiler_params=pltpu.CompilerParams(needs_layout_passes=False),
    scratch_types=[
        pltpu.VMEM((N,), jnp.float32),
        pltpu.VMEM((N,), jnp.float32),
        pltpu.VMEM((N,), jnp.float32),
        pltpu.VMEM((N,), jnp.int32),
        pltpu.VMEM((N,), jnp.int32),
        pltpu.VMEM((L,), jnp.float32),
    ],
)
def _listmle_sc(scores_hbm, labels_hbm, out_hbm, sv, lv, ev, uav, umv, outv):
    _body(scores_hbm, labels_hbm, out_hbm, sv, lv, ev, uav, umv, outv)


def kernel(scores, labels):
    partials = _listmle_sc(scores, labels)
    return jnp.sum(partials[:, 0])
